# baseline (speedup 1.0000x reference)
import functools

import jax
import jax.numpy as jnp
from jax import lax
from jax.experimental import pallas as pl
from jax.experimental.pallas import tpu as pltpu
from jax.experimental.pallas import tpu_sc as plsc

TOKENS = 32768
CH = 256
LANES = 16
VECS = CH // LANES
NC, NS = 2, 16
NW = NC * NS

SC_ROWS = 6144
TC_ROWS = TOKENS - SC_ROWS
ROWS_PER_W = SC_ROWS // NW
CHUNK = 64
NCHUNK = ROWS_PER_W // CHUNK

TC_BLOCK = 4096

_mesh = plsc.VectorSubcoreMesh(core_axis_name="c", subcore_axis_name="s")


@functools.partial(
    pl.kernel,
    mesh=_mesh,
    out_type=jax.ShapeDtypeStruct((SC_ROWS, CH), jnp.float32),
    scratch_types=[
        pltpu.VMEM((CH,), jnp.float32),
        pltpu.VMEM((2, CHUNK, CH), jnp.float32),
        pltpu.VMEM((2, CHUNK, CH), jnp.float32),
        pltpu.VMEM((2, CHUNK, CH), jnp.float32),
        pltpu.SemaphoreType.DMA,
        pltpu.SemaphoreType.DMA,
    ],
)
def _sc_blend(x1_hbm, x2_hbm, gate_hbm, out_hbm, g_v, a_v, b_v, o_v,
              in_sem, out_sem):
    wid = lax.axis_index("s") * NC + lax.axis_index("c")
    in_base = TC_ROWS + wid * ROWS_PER_W
    out_base = wid * ROWS_PER_W

    pltpu.sync_copy(gate_hbm, g_v)
    sig = [
        1.0 / (1.0 + jnp.exp(-g_v[pl.ds(LANES * j, LANES)]))
        for j in range(VECS)
    ]

    def start_in(ci, b):
        row0 = in_base + ci * CHUNK
        pltpu.make_async_copy(
            x1_hbm.at[pl.ds(row0, CHUNK)], a_v.at[b], in_sem).start()
        pltpu.make_async_copy(
            x2_hbm.at[pl.ds(row0, CHUNK)], b_v.at[b], in_sem).start()

    def wait_in(b):
        pltpu.make_async_copy(
            x1_hbm.at[pl.ds(in_base, CHUNK)], a_v.at[b], in_sem).wait()
        pltpu.make_async_copy(
            x2_hbm.at[pl.ds(in_base, CHUNK)], b_v.at[b], in_sem).wait()

    def start_out(ci, b):
        row0 = out_base + ci * CHUNK
        pltpu.make_async_copy(
            o_v.at[b], out_hbm.at[pl.ds(row0, CHUNK)], out_sem).start()

    def wait_out_one(b):
        pltpu.make_async_copy(
            o_v.at[b], out_hbm.at[pl.ds(out_base, CHUNK)], out_sem).wait()

    def compute(b):
        def row_body(r, c2):
            for j in range(VECS):
                sl = pl.ds(LANES * j, LANES)
                x1 = a_v[b, r, sl]
                x2 = b_v[b, r, sl]
                o_v[b, r, sl] = x2 + sig[j] * (x1 - x2)
            return c2

        lax.fori_loop(0, CHUNK, row_body, 0)

    start_in(0, 0)
    for ci in range(NCHUNK):
        b = ci % 2
        if ci + 1 < NCHUNK:
            if ci >= 1:
                wait_out_one(1 - b)
            start_in(ci + 1, 1 - b)
        wait_in(b)
        compute(b)
        start_out(ci, b)
    if NCHUNK >= 2:
        wait_out_one(1)
    wait_out_one(0)


TC_STEP = 1024
TC_NSTEP = TC_ROWS // TC_STEP
TC_NBUF = 6


def _tc_blend(x1_hbm, x2_hbm, gate_ref, out_hbm, a_v, b_v, o_v,
              in_sems, out_sems):
    g = jax.nn.sigmoid(gate_ref[...])

    def start_in(s):
        buf = s % TC_NBUF
        row0 = s * TC_STEP
        pltpu.make_async_copy(
            x1_hbm.at[pl.ds(row0, TC_STEP)], a_v.at[buf],
            in_sems.at[buf]).start()
        pltpu.make_async_copy(
            x2_hbm.at[pl.ds(row0, TC_STEP)], b_v.at[buf],
            in_sems.at[buf]).start()

    def wait_in(s):
        buf = s % TC_NBUF
        pltpu.make_async_copy(
            x1_hbm.at[pl.ds(0, TC_STEP)], a_v.at[buf],
            in_sems.at[buf]).wait()
        pltpu.make_async_copy(
            x2_hbm.at[pl.ds(0, TC_STEP)], b_v.at[buf],
            in_sems.at[buf]).wait()

    def start_out(s):
        buf = s % TC_NBUF
        row0 = s * TC_STEP
        pltpu.make_async_copy(
            o_v.at[buf], out_hbm.at[pl.ds(row0, TC_STEP)],
            out_sems.at[buf]).start()

    def wait_out(s):
        buf = s % TC_NBUF
        pltpu.make_async_copy(
            o_v.at[buf], out_hbm.at[pl.ds(0, TC_STEP)],
            out_sems.at[buf]).wait()

    for s in range(min(TC_NBUF, TC_NSTEP)):
        start_in(s)
    for s in range(TC_NSTEP):
        buf = s % TC_NBUF
        wait_in(s)
        if s >= TC_NBUF:
            wait_out(s - TC_NBUF)
        o_v[buf] = b_v[buf] + g * (a_v[buf] - b_v[buf])
        start_out(s)
        if s + TC_NBUF < TC_NSTEP:
            start_in(s + TC_NBUF)
    for s in range(max(TC_NSTEP - TC_NBUF, 0), TC_NSTEP):
        wait_out(s)


_tc_call = pl.pallas_call(
    _tc_blend,
    in_specs=[
        pl.BlockSpec(memory_space=pl.ANY),
        pl.BlockSpec(memory_space=pl.ANY),
        pl.BlockSpec((1, CH), lambda: (0, 0)),
    ],
    out_specs=pl.BlockSpec(memory_space=pl.ANY),
    out_shape=jax.ShapeDtypeStruct((TOKENS, CH), jnp.float32),
    scratch_shapes=[
        pltpu.VMEM((TC_NBUF, TC_STEP, CH), jnp.float32),
        pltpu.VMEM((TC_NBUF, TC_STEP, CH), jnp.float32),
        pltpu.VMEM((TC_NBUF, TC_STEP, CH), jnp.float32),
        pltpu.SemaphoreType.DMA((TC_NBUF,)),
        pltpu.SemaphoreType.DMA((TC_NBUF,)),
    ],
)


def kernel(x_1, x_2, gate):
    sc_part = _sc_blend(x_1, x_2, gate)
    tc_full = _tc_call(x_1, x_2, gate.reshape(1, CH))
    return lax.dynamic_update_slice(tc_full, sc_part, (TC_ROWS, 0))


# speedup vs baseline: 1.0725x; 1.0725x over previous
import functools

import jax
import jax.numpy as jnp
from jax import lax
from jax.experimental import pallas as pl
from jax.experimental.pallas import tpu as pltpu
from jax.experimental.pallas import tpu_sc as plsc

TOKENS = 32768
CH = 256
LANES = 16
VECS = CH // LANES
NC, NS = 1, 16
NW = NC * NS

SC_ROWS = 4096
TC_ROWS = TOKENS - SC_ROWS
ROWS_PER_W = SC_ROWS // NW
CHUNK = 64
NCHUNK = ROWS_PER_W // CHUNK

TC_BLOCK = 4096

_mesh = plsc.VectorSubcoreMesh(core_axis_name="c", subcore_axis_name="s", num_cores=NC)


@functools.partial(
    pl.kernel,
    mesh=_mesh,
    out_type=jax.ShapeDtypeStruct((SC_ROWS, CH), jnp.float32),
    scratch_types=[
        pltpu.VMEM((CH,), jnp.float32),
        pltpu.VMEM((2, CHUNK, CH), jnp.float32),
        pltpu.VMEM((2, CHUNK, CH), jnp.float32),
        pltpu.VMEM((2, CHUNK, CH), jnp.float32),
        pltpu.SemaphoreType.DMA,
        pltpu.SemaphoreType.DMA,
    ],
)
def _sc_blend(x1_hbm, x2_hbm, gate_hbm, out_hbm, g_v, a_v, b_v, o_v,
              in_sem, out_sem):
    wid = lax.axis_index("s") * NC + lax.axis_index("c")
    in_base = TC_ROWS + wid * ROWS_PER_W
    out_base = wid * ROWS_PER_W

    pltpu.sync_copy(gate_hbm, g_v)
    sig = [
        1.0 / (1.0 + jnp.exp(-g_v[pl.ds(LANES * j, LANES)]))
        for j in range(VECS)
    ]

    def start_in(ci, b):
        row0 = in_base + ci * CHUNK
        pltpu.make_async_copy(
            x1_hbm.at[pl.ds(row0, CHUNK)], a_v.at[b], in_sem).start()
        pltpu.make_async_copy(
            x2_hbm.at[pl.ds(row0, CHUNK)], b_v.at[b], in_sem).start()

    def wait_in(b):
        pltpu.make_async_copy(
            x1_hbm.at[pl.ds(in_base, CHUNK)], a_v.at[b], in_sem).wait()
        pltpu.make_async_copy(
            x2_hbm.at[pl.ds(in_base, CHUNK)], b_v.at[b], in_sem).wait()

    def start_out(ci, b):
        row0 = out_base + ci * CHUNK
        pltpu.make_async_copy(
            o_v.at[b], out_hbm.at[pl.ds(row0, CHUNK)], out_sem).start()

    def wait_out_one(b):
        pltpu.make_async_copy(
            o_v.at[b], out_hbm.at[pl.ds(out_base, CHUNK)], out_sem).wait()

    def compute(b):
        def row_body(r, c2):
            for j in range(VECS):
                sl = pl.ds(LANES * j, LANES)
                x1 = a_v[b, r, sl]
                x2 = b_v[b, r, sl]
                o_v[b, r, sl] = x2 + sig[j] * (x1 - x2)
            return c2

        lax.fori_loop(0, CHUNK, row_body, 0)

    start_in(0, 0)
    for ci in range(NCHUNK):
        b = ci % 2
        if ci + 1 < NCHUNK:
            if ci >= 1:
                wait_out_one(1 - b)
            start_in(ci + 1, 1 - b)
        wait_in(b)
        compute(b)
        start_out(ci, b)
    if NCHUNK >= 2:
        wait_out_one(1)
    wait_out_one(0)


TC_STEP = 1024
TC_NSTEP = TC_ROWS // TC_STEP
TC_NBUF = 6


def _tc_blend(x1_hbm, x2_hbm, gate_ref, out_hbm, a_v, b_v, o_v,
              in_sems, out_sems):
    g = jax.nn.sigmoid(gate_ref[...])

    def start_in(s):
        buf = s % TC_NBUF
        row0 = s * TC_STEP
        pltpu.make_async_copy(
            x1_hbm.at[pl.ds(row0, TC_STEP)], a_v.at[buf],
            in_sems.at[buf]).start()
        pltpu.make_async_copy(
            x2_hbm.at[pl.ds(row0, TC_STEP)], b_v.at[buf],
            in_sems.at[buf]).start()

    def wait_in(s):
        buf = s % TC_NBUF
        pltpu.make_async_copy(
            x1_hbm.at[pl.ds(0, TC_STEP)], a_v.at[buf],
            in_sems.at[buf]).wait()
        pltpu.make_async_copy(
            x2_hbm.at[pl.ds(0, TC_STEP)], b_v.at[buf],
            in_sems.at[buf]).wait()

    def start_out(s):
        buf = s % TC_NBUF
        row0 = s * TC_STEP
        pltpu.make_async_copy(
            o_v.at[buf], out_hbm.at[pl.ds(row0, TC_STEP)],
            out_sems.at[buf]).start()

    def wait_out(s):
        buf = s % TC_NBUF
        pltpu.make_async_copy(
            o_v.at[buf], out_hbm.at[pl.ds(0, TC_STEP)],
            out_sems.at[buf]).wait()

    for s in range(min(TC_NBUF, TC_NSTEP)):
        start_in(s)
    for s in range(TC_NSTEP):
        buf = s % TC_NBUF
        wait_in(s)
        if s >= TC_NBUF:
            wait_out(s - TC_NBUF)
        o_v[buf] = b_v[buf] + g * (a_v[buf] - b_v[buf])
        start_out(s)
        if s + TC_NBUF < TC_NSTEP:
            start_in(s + TC_NBUF)
    for s in range(max(TC_NSTEP - TC_NBUF, 0), TC_NSTEP):
        wait_out(s)


_tc_call = pl.pallas_call(
    _tc_blend,
    in_specs=[
        pl.BlockSpec(memory_space=pl.ANY),
        pl.BlockSpec(memory_space=pl.ANY),
        pl.BlockSpec((1, CH), lambda: (0, 0)),
    ],
    out_specs=pl.BlockSpec(memory_space=pl.ANY),
    out_shape=jax.ShapeDtypeStruct((TOKENS, CH), jnp.float32),
    scratch_shapes=[
        pltpu.VMEM((TC_NBUF, TC_STEP, CH), jnp.float32),
        pltpu.VMEM((TC_NBUF, TC_STEP, CH), jnp.float32),
        pltpu.VMEM((TC_NBUF, TC_STEP, CH), jnp.float32),
        pltpu.SemaphoreType.DMA((TC_NBUF,)),
        pltpu.SemaphoreType.DMA((TC_NBUF,)),
    ],
)


def kernel(x_1, x_2, gate):
    sc_part = _sc_blend(x_1, x_2, gate)
    tc_full = _tc_call(x_1, x_2, gate.reshape(1, CH))
    return lax.dynamic_update_slice(tc_full, sc_part, (TC_ROWS, 0))


# speedup vs baseline: 1.0968x; 1.0227x over previous
import functools

import jax
import jax.numpy as jnp
from jax import lax
from jax.experimental import pallas as pl
from jax.experimental.pallas import tpu as pltpu
from jax.experimental.pallas import tpu_sc as plsc

TOKENS = 32768
CH = 256
LANES = 16
VECS = CH // LANES
NC, NS = 1, 16
NW = NC * NS

SC_ROWS = 2048
TC_ROWS = TOKENS - SC_ROWS
ROWS_PER_W = SC_ROWS // NW
CHUNK = 64
NCHUNK = ROWS_PER_W // CHUNK

TC_BLOCK = 4096

_mesh = plsc.VectorSubcoreMesh(core_axis_name="c", subcore_axis_name="s", num_cores=NC)


@functools.partial(
    pl.kernel,
    mesh=_mesh,
    out_type=jax.ShapeDtypeStruct((SC_ROWS, CH), jnp.float32),
    scratch_types=[
        pltpu.VMEM((CH,), jnp.float32),
        pltpu.VMEM((2, CHUNK, CH), jnp.float32),
        pltpu.VMEM((2, CHUNK, CH), jnp.float32),
        pltpu.VMEM((2, CHUNK, CH), jnp.float32),
        pltpu.SemaphoreType.DMA,
        pltpu.SemaphoreType.DMA,
    ],
)
def _sc_blend(x1_hbm, x2_hbm, gate_hbm, out_hbm, g_v, a_v, b_v, o_v,
              in_sem, out_sem):
    wid = lax.axis_index("s") * NC + lax.axis_index("c")
    in_base = TC_ROWS + wid * ROWS_PER_W
    out_base = wid * ROWS_PER_W

    pltpu.sync_copy(gate_hbm, g_v)
    sig = [
        1.0 / (1.0 + jnp.exp(-g_v[pl.ds(LANES * j, LANES)]))
        for j in range(VECS)
    ]

    def start_in(ci, b):
        row0 = in_base + ci * CHUNK
        pltpu.make_async_copy(
            x1_hbm.at[pl.ds(row0, CHUNK)], a_v.at[b], in_sem).start()
        pltpu.make_async_copy(
            x2_hbm.at[pl.ds(row0, CHUNK)], b_v.at[b], in_sem).start()

    def wait_in(b):
        pltpu.make_async_copy(
            x1_hbm.at[pl.ds(in_base, CHUNK)], a_v.at[b], in_sem).wait()
        pltpu.make_async_copy(
            x2_hbm.at[pl.ds(in_base, CHUNK)], b_v.at[b], in_sem).wait()

    def start_out(ci, b):
        row0 = out_base + ci * CHUNK
        pltpu.make_async_copy(
            o_v.at[b], out_hbm.at[pl.ds(row0, CHUNK)], out_sem).start()

    def wait_out_one(b):
        pltpu.make_async_copy(
            o_v.at[b], out_hbm.at[pl.ds(out_base, CHUNK)], out_sem).wait()

    def compute(b):
        def row_body(r, c2):
            for j in range(VECS):
                sl = pl.ds(LANES * j, LANES)
                x1 = a_v[b, r, sl]
                x2 = b_v[b, r, sl]
                o_v[b, r, sl] = x2 + sig[j] * (x1 - x2)
            return c2

        lax.fori_loop(0, CHUNK, row_body, 0)

    start_in(0, 0)
    for ci in range(NCHUNK):
        b = ci % 2
        if ci + 1 < NCHUNK:
            if ci >= 1:
                wait_out_one(1 - b)
            start_in(ci + 1, 1 - b)
        wait_in(b)
        compute(b)
        start_out(ci, b)
    if NCHUNK >= 2:
        wait_out_one(1)
    wait_out_one(0)


TC_STEP = 1024
TC_NSTEP = TC_ROWS // TC_STEP
TC_NBUF = 6


def _tc_blend(x1_hbm, x2_hbm, gate_ref, out_hbm, a_v, b_v, o_v,
              in_sems, out_sems):
    g = jax.nn.sigmoid(gate_ref[...])

    def start_in(s):
        buf = s % TC_NBUF
        row0 = s * TC_STEP
        pltpu.make_async_copy(
            x1_hbm.at[pl.ds(row0, TC_STEP)], a_v.at[buf],
            in_sems.at[buf]).start()
        pltpu.make_async_copy(
            x2_hbm.at[pl.ds(row0, TC_STEP)], b_v.at[buf],
            in_sems.at[buf]).start()

    def wait_in(s):
        buf = s % TC_NBUF
        pltpu.make_async_copy(
            x1_hbm.at[pl.ds(0, TC_STEP)], a_v.at[buf],
            in_sems.at[buf]).wait()
        pltpu.make_async_copy(
            x2_hbm.at[pl.ds(0, TC_STEP)], b_v.at[buf],
            in_sems.at[buf]).wait()

    def start_out(s):
        buf = s % TC_NBUF
        row0 = s * TC_STEP
        pltpu.make_async_copy(
            o_v.at[buf], out_hbm.at[pl.ds(row0, TC_STEP)],
            out_sems.at[buf]).start()

    def wait_out(s):
        buf = s % TC_NBUF
        pltpu.make_async_copy(
            o_v.at[buf], out_hbm.at[pl.ds(0, TC_STEP)],
            out_sems.at[buf]).wait()

    for s in range(min(TC_NBUF, TC_NSTEP)):
        start_in(s)
    for s in range(TC_NSTEP):
        buf = s % TC_NBUF
        wait_in(s)
        if s >= TC_NBUF:
            wait_out(s - TC_NBUF)
        o_v[buf] = b_v[buf] + g * (a_v[buf] - b_v[buf])
        start_out(s)
        if s + TC_NBUF < TC_NSTEP:
            start_in(s + TC_NBUF)
    for s in range(max(TC_NSTEP - TC_NBUF, 0), TC_NSTEP):
        wait_out(s)


_tc_call = pl.pallas_call(
    _tc_blend,
    in_specs=[
        pl.BlockSpec(memory_space=pl.ANY),
        pl.BlockSpec(memory_space=pl.ANY),
        pl.BlockSpec((1, CH), lambda: (0, 0)),
    ],
    out_specs=pl.BlockSpec(memory_space=pl.ANY),
    out_shape=jax.ShapeDtypeStruct((TOKENS, CH), jnp.float32),
    scratch_shapes=[
        pltpu.VMEM((TC_NBUF, TC_STEP, CH), jnp.float32),
        pltpu.VMEM((TC_NBUF, TC_STEP, CH), jnp.float32),
        pltpu.VMEM((TC_NBUF, TC_STEP, CH), jnp.float32),
        pltpu.SemaphoreType.DMA((TC_NBUF,)),
        pltpu.SemaphoreType.DMA((TC_NBUF,)),
    ],
)


def kernel(x_1, x_2, gate):
    sc_part = _sc_blend(x_1, x_2, gate)
    tc_full = _tc_call(x_1, x_2, gate.reshape(1, CH))
    return lax.dynamic_update_slice(tc_full, sc_part, (TC_ROWS, 0))


# speedup vs baseline: 1.1227x; 1.0236x over previous
import functools

import jax
import jax.numpy as jnp
from jax import lax
from jax.experimental import pallas as pl
from jax.experimental.pallas import tpu as pltpu
from jax.experimental.pallas import tpu_sc as plsc

TOKENS = 32768
CH = 256
LANES = 16
VECS = CH // LANES
NC, NS = 1, 16
NW = NC * NS

SC_ROWS = 1024
TC_ROWS = TOKENS - SC_ROWS
ROWS_PER_W = SC_ROWS // NW
CHUNK = 64
NCHUNK = ROWS_PER_W // CHUNK

TC_BLOCK = 4096

_mesh = plsc.VectorSubcoreMesh(core_axis_name="c", subcore_axis_name="s", num_cores=NC)


@functools.partial(
    pl.kernel,
    mesh=_mesh,
    out_type=jax.ShapeDtypeStruct((SC_ROWS, CH), jnp.float32),
    scratch_types=[
        pltpu.VMEM((CH,), jnp.float32),
        pltpu.VMEM((2, CHUNK, CH), jnp.float32),
        pltpu.VMEM((2, CHUNK, CH), jnp.float32),
        pltpu.VMEM((2, CHUNK, CH), jnp.float32),
        pltpu.SemaphoreType.DMA,
        pltpu.SemaphoreType.DMA,
    ],
)
def _sc_blend(x1_hbm, x2_hbm, gate_hbm, out_hbm, g_v, a_v, b_v, o_v,
              in_sem, out_sem):
    wid = lax.axis_index("s") * NC + lax.axis_index("c")
    in_base = TC_ROWS + wid * ROWS_PER_W
    out_base = wid * ROWS_PER_W

    pltpu.sync_copy(gate_hbm, g_v)
    sig = [
        1.0 / (1.0 + jnp.exp(-g_v[pl.ds(LANES * j, LANES)]))
        for j in range(VECS)
    ]

    def start_in(ci, b):
        row0 = in_base + ci * CHUNK
        pltpu.make_async_copy(
            x1_hbm.at[pl.ds(row0, CHUNK)], a_v.at[b], in_sem).start()
        pltpu.make_async_copy(
            x2_hbm.at[pl.ds(row0, CHUNK)], b_v.at[b], in_sem).start()

    def wait_in(b):
        pltpu.make_async_copy(
            x1_hbm.at[pl.ds(in_base, CHUNK)], a_v.at[b], in_sem).wait()
        pltpu.make_async_copy(
            x2_hbm.at[pl.ds(in_base, CHUNK)], b_v.at[b], in_sem).wait()

    def start_out(ci, b):
        row0 = out_base + ci * CHUNK
        pltpu.make_async_copy(
            o_v.at[b], out_hbm.at[pl.ds(row0, CHUNK)], out_sem).start()

    def wait_out_one(b):
        pltpu.make_async_copy(
            o_v.at[b], out_hbm.at[pl.ds(out_base, CHUNK)], out_sem).wait()

    def compute(b):
        def row_body(r, c2):
            for j in range(VECS):
                sl = pl.ds(LANES * j, LANES)
                x1 = a_v[b, r, sl]
                x2 = b_v[b, r, sl]
                o_v[b, r, sl] = x2 + sig[j] * (x1 - x2)
            return c2

        lax.fori_loop(0, CHUNK, row_body, 0)

    start_in(0, 0)
    for ci in range(NCHUNK):
        b = ci % 2
        if ci + 1 < NCHUNK:
            if ci >= 1:
                wait_out_one(1 - b)
            start_in(ci + 1, 1 - b)
        wait_in(b)
        compute(b)
        start_out(ci, b)
    if NCHUNK >= 2:
        wait_out_one(1)
    wait_out_one(0)


TC_STEP = 1024
TC_NSTEP = TC_ROWS // TC_STEP
TC_NBUF = 6


def _tc_blend(x1_hbm, x2_hbm, gate_ref, out_hbm, a_v, b_v, o_v,
              in_sems, out_sems):
    g = jax.nn.sigmoid(gate_ref[...])

    def start_in(s):
        buf = s % TC_NBUF
        row0 = s * TC_STEP
        pltpu.make_async_copy(
            x1_hbm.at[pl.ds(row0, TC_STEP)], a_v.at[buf],
            in_sems.at[buf]).start()
        pltpu.make_async_copy(
            x2_hbm.at[pl.ds(row0, TC_STEP)], b_v.at[buf],
            in_sems.at[buf]).start()

    def wait_in(s):
        buf = s % TC_NBUF
        pltpu.make_async_copy(
            x1_hbm.at[pl.ds(0, TC_STEP)], a_v.at[buf],
            in_sems.at[buf]).wait()
        pltpu.make_async_copy(
            x2_hbm.at[pl.ds(0, TC_STEP)], b_v.at[buf],
            in_sems.at[buf]).wait()

    def start_out(s):
        buf = s % TC_NBUF
        row0 = s * TC_STEP
        pltpu.make_async_copy(
            o_v.at[buf], out_hbm.at[pl.ds(row0, TC_STEP)],
            out_sems.at[buf]).start()

    def wait_out(s):
        buf = s % TC_NBUF
        pltpu.make_async_copy(
            o_v.at[buf], out_hbm.at[pl.ds(0, TC_STEP)],
            out_sems.at[buf]).wait()

    for s in range(min(TC_NBUF, TC_NSTEP)):
        start_in(s)
    for s in range(TC_NSTEP):
        buf = s % TC_NBUF
        wait_in(s)
        if s >= TC_NBUF:
            wait_out(s - TC_NBUF)
        o_v[buf] = b_v[buf] + g * (a_v[buf] - b_v[buf])
        start_out(s)
        if s + TC_NBUF < TC_NSTEP:
            start_in(s + TC_NBUF)
    for s in range(max(TC_NSTEP - TC_NBUF, 0), TC_NSTEP):
        wait_out(s)


_tc_call = pl.pallas_call(
    _tc_blend,
    in_specs=[
        pl.BlockSpec(memory_space=pl.ANY),
        pl.BlockSpec(memory_space=pl.ANY),
        pl.BlockSpec((1, CH), lambda: (0, 0)),
    ],
    out_specs=pl.BlockSpec(memory_space=pl.ANY),
    out_shape=jax.ShapeDtypeStruct((TOKENS, CH), jnp.float32),
    scratch_shapes=[
        pltpu.VMEM((TC_NBUF, TC_STEP, CH), jnp.float32),
        pltpu.VMEM((TC_NBUF, TC_STEP, CH), jnp.float32),
        pltpu.VMEM((TC_NBUF, TC_STEP, CH), jnp.float32),
        pltpu.SemaphoreType.DMA((TC_NBUF,)),
        pltpu.SemaphoreType.DMA((TC_NBUF,)),
    ],
)


def kernel(x_1, x_2, gate):
    sc_part = _sc_blend(x_1, x_2, gate)
    tc_full = _tc_call(x_1, x_2, gate.reshape(1, CH))
    return lax.dynamic_update_slice(tc_full, sc_part, (TC_ROWS, 0))


# speedup vs baseline: 1.1228x; 1.0000x over previous
import functools

import jax
import jax.numpy as jnp
from jax import lax
from jax.experimental import pallas as pl
from jax.experimental.pallas import tpu as pltpu
from jax.experimental.pallas import tpu_sc as plsc

TOKENS = 32768
CH = 256
LANES = 16
VECS = CH // LANES
NC, NS = 1, 16
NW = NC * NS

SC_ROWS = 1024
TC_ROWS = TOKENS - SC_ROWS
ROWS_PER_W = SC_ROWS // NW
CHUNK = 64
NCHUNK = ROWS_PER_W // CHUNK

TC_BLOCK = 4096

_mesh = plsc.VectorSubcoreMesh(core_axis_name="c", subcore_axis_name="s", num_cores=NC)


@functools.partial(
    pl.kernel,
    mesh=_mesh,
    out_type=jax.ShapeDtypeStruct((SC_ROWS, CH), jnp.float32),
    scratch_types=[
        pltpu.VMEM((CH,), jnp.float32),
        pltpu.VMEM((2, CHUNK, CH), jnp.float32),
        pltpu.VMEM((2, CHUNK, CH), jnp.float32),
        pltpu.VMEM((2, CHUNK, CH), jnp.float32),
        pltpu.SemaphoreType.DMA,
        pltpu.SemaphoreType.DMA,
    ],
)
def _sc_blend(x1_hbm, x2_hbm, gate_hbm, out_hbm, g_v, a_v, b_v, o_v,
              in_sem, out_sem):
    wid = lax.axis_index("s") * NC + lax.axis_index("c")
    in_base = TC_ROWS + wid * ROWS_PER_W
    out_base = wid * ROWS_PER_W

    pltpu.sync_copy(gate_hbm, g_v)
    sig = [
        1.0 / (1.0 + jnp.exp(-g_v[pl.ds(LANES * j, LANES)]))
        for j in range(VECS)
    ]

    def start_in(ci, b):
        row0 = in_base + ci * CHUNK
        pltpu.make_async_copy(
            x1_hbm.at[pl.ds(row0, CHUNK)], a_v.at[b], in_sem).start()
        pltpu.make_async_copy(
            x2_hbm.at[pl.ds(row0, CHUNK)], b_v.at[b], in_sem).start()

    def wait_in(b):
        pltpu.make_async_copy(
            x1_hbm.at[pl.ds(in_base, CHUNK)], a_v.at[b], in_sem).wait()
        pltpu.make_async_copy(
            x2_hbm.at[pl.ds(in_base, CHUNK)], b_v.at[b], in_sem).wait()

    def start_out(ci, b):
        row0 = out_base + ci * CHUNK
        pltpu.make_async_copy(
            o_v.at[b], out_hbm.at[pl.ds(row0, CHUNK)], out_sem).start()

    def wait_out_one(b):
        pltpu.make_async_copy(
            o_v.at[b], out_hbm.at[pl.ds(out_base, CHUNK)], out_sem).wait()

    def compute(b):
        def row_body(r, c2):
            for j in range(VECS):
                sl = pl.ds(LANES * j, LANES)
                x1 = a_v[b, r, sl]
                x2 = b_v[b, r, sl]
                o_v[b, r, sl] = x2 + sig[j] * (x1 - x2)
            return c2

        lax.fori_loop(0, CHUNK, row_body, 0)

    start_in(0, 0)
    for ci in range(NCHUNK):
        b = ci % 2
        if ci + 1 < NCHUNK:
            if ci >= 1:
                wait_out_one(1 - b)
            start_in(ci + 1, 1 - b)
        wait_in(b)
        compute(b)
        start_out(ci, b)
    if NCHUNK >= 2:
        wait_out_one(1)
    wait_out_one(0)


TC_STEP = 512
TC_NSTEP = TC_ROWS // TC_STEP
TC_NBUF = 12


def _tc_blend(x1_hbm, x2_hbm, gate_ref, out_hbm, a_v, b_v, o_v,
              in_sems, out_sems):
    g = jax.nn.sigmoid(gate_ref[...])

    def start_in(s):
        buf = s % TC_NBUF
        row0 = s * TC_STEP
        pltpu.make_async_copy(
            x1_hbm.at[pl.ds(row0, TC_STEP)], a_v.at[buf],
            in_sems.at[buf]).start()
        pltpu.make_async_copy(
            x2_hbm.at[pl.ds(row0, TC_STEP)], b_v.at[buf],
            in_sems.at[buf]).start()

    def wait_in(s):
        buf = s % TC_NBUF
        pltpu.make_async_copy(
            x1_hbm.at[pl.ds(0, TC_STEP)], a_v.at[buf],
            in_sems.at[buf]).wait()
        pltpu.make_async_copy(
            x2_hbm.at[pl.ds(0, TC_STEP)], b_v.at[buf],
            in_sems.at[buf]).wait()

    def start_out(s):
        buf = s % TC_NBUF
        row0 = s * TC_STEP
        pltpu.make_async_copy(
            o_v.at[buf], out_hbm.at[pl.ds(row0, TC_STEP)],
            out_sems.at[buf]).start()

    def wait_out(s):
        buf = s % TC_NBUF
        pltpu.make_async_copy(
            o_v.at[buf], out_hbm.at[pl.ds(0, TC_STEP)],
            out_sems.at[buf]).wait()

    for s in range(min(TC_NBUF, TC_NSTEP)):
        start_in(s)
    for s in range(TC_NSTEP):
        buf = s % TC_NBUF
        wait_in(s)
        if s >= TC_NBUF:
            wait_out(s - TC_NBUF)
        o_v[buf] = b_v[buf] + g * (a_v[buf] - b_v[buf])
        start_out(s)
        if s + TC_NBUF < TC_NSTEP:
            start_in(s + TC_NBUF)
    for s in range(max(TC_NSTEP - TC_NBUF, 0), TC_NSTEP):
        wait_out(s)


_tc_call = pl.pallas_call(
    _tc_blend,
    in_specs=[
        pl.BlockSpec(memory_space=pl.ANY),
        pl.BlockSpec(memory_space=pl.ANY),
        pl.BlockSpec((1, CH), lambda: (0, 0)),
    ],
    out_specs=pl.BlockSpec(memory_space=pl.ANY),
    out_shape=jax.ShapeDtypeStruct((TOKENS, CH), jnp.float32),
    scratch_shapes=[
        pltpu.VMEM((TC_NBUF, TC_STEP, CH), jnp.float32),
        pltpu.VMEM((TC_NBUF, TC_STEP, CH), jnp.float32),
        pltpu.VMEM((TC_NBUF, TC_STEP, CH), jnp.float32),
        pltpu.SemaphoreType.DMA((TC_NBUF,)),
        pltpu.SemaphoreType.DMA((TC_NBUF,)),
    ],
)


def kernel(x_1, x_2, gate):
    sc_part = _sc_blend(x_1, x_2, gate)
    tc_full = _tc_call(x_1, x_2, gate.reshape(1, CH))
    return lax.dynamic_update_slice(tc_full, sc_part, (TC_ROWS, 0))
